# baseline (speedup 1.0000x reference)
import functools

import jax
import jax.numpy as jnp
from jax import lax
from jax.experimental import pallas as pl
from jax.experimental.pallas import tpu as pltpu
from jax.experimental.pallas import tpu_sc as plsc

_BT = 128
_D = 64


@functools.lru_cache(maxsize=None)
def _make_gather(NB, SEQ, V):
    info = plsc.get_sparse_core_info()
    NC, NS = info.num_cores, info.num_subcores
    NW = NC * NS
    assert NB % (NW * _BT) == 0
    bt_per_w = NB // _BT // NW
    n_slabs = bt_per_w * SEQ
    NBT = NB // _BT

    mesh = plsc.VectorSubcoreMesh(core_axis_name="c", subcore_axis_name="s")

    @functools.partial(
        pl.kernel,
        mesh=mesh,
        out_type=jax.ShapeDtypeStruct((SEQ, _D // 8, NBT, 8 * _BT), jnp.float32),
        compiler_params=pltpu.CompilerParams(
            use_tc_tiling_on_sc=False, needs_layout_passes=False
        ),
        scratch_types=[
            pltpu.VMEM((2, _BT), jnp.int32),
            pltpu.VMEM((2, _BT, _D), jnp.float32),
            pltpu.VMEM((2, _D // 8, 8 * _BT), jnp.float32),
            pltpu.SemaphoreType.DMA,
            pltpu.SemaphoreType.DMA,
            pltpu.SemaphoreType.DMA,
            pltpu.SemaphoreType.DMA,
            pltpu.SemaphoreType.DMA,
            pltpu.SemaphoreType.DMA,
        ],
    )
    def gather_kernel(table_hbm, idx_hbm, out_hbm, sidx_v, rows_v, slab_v,
                      i0, i1, g0, g1, w0, w1):
        wid = lax.axis_index("s") * NC + lax.axis_index("c")
        bt0 = wid * bt_per_w
        isems = (i0, i1)
        gsems = (g0, g1)
        wsems = (w0, w1)
        iota = lax.iota(jnp.int32, 16)
        iotas = [iota + 16 * m for m in range(8)]

        def slab_pos(n):
            bt_local = n // SEQ
            s = n - bt_local * SEQ
            return bt0 + bt_local, s

        def idx_slice(n):
            bt, s = slab_pos(n)
            return idx_hbm.at[pl.ds(s * NB + bt * _BT, _BT)]

        def start_idx(n, buf):
            pltpu.async_copy(idx_slice(n), sidx_v.at[buf], isems[buf])

        def wait_idx(n, buf):
            pltpu.make_async_copy(idx_slice(n), sidx_v.at[buf], isems[buf]).wait()

        def start_gather(buf):
            pltpu.async_copy(table_hbm.at[sidx_v.at[buf]], rows_v.at[buf], gsems[buf])

        def wait_gather(buf):
            pltpu.make_async_copy(
                table_hbm.at[sidx_v.at[buf]], rows_v.at[buf], gsems[buf]
            ).wait()

        def out_slice(n):
            bt, s = slab_pos(n)
            return out_hbm.at[s, :, bt, :]

        def start_write(n, buf):
            pltpu.async_copy(slab_v.at[buf], out_slice(n), wsems[buf])

        def wait_write(n, buf):
            pltpu.make_async_copy(slab_v.at[buf], out_slice(n), wsems[buf]).wait()

        def transpose(buf):
            rbuf = rows_v.at[buf]

            @pl.loop(0, _D // 8)
            def _(jt):
                for jr in range(8):
                    jcol = jnp.zeros((16,), jnp.int32) + (jt * 8 + jr)
                    for m in range(8):
                        vec = plsc.load_gather(rbuf, [iotas[m], jcol])
                        slab_v[buf, jt, pl.ds(jr * _BT + m * 16, 16)] = vec

        start_idx(0, 0)
        start_idx(1, 1)
        wait_idx(0, 0)
        start_gather(0)

        @pl.loop(0, n_slabs, step=2)
        def _(n0):
            for b in range(2):
                n = n0 + b
                nb = 1 - b
                wait_gather(b)

                @pl.when(n + 2 < n_slabs)
                def _():
                    start_idx(n + 2, b)

                @pl.when(n + 1 < n_slabs)
                def _():
                    wait_idx(n + 1, nb)
                    start_gather(nb)

                @pl.when(n >= 2)
                def _():
                    wait_write(n - 2, b)

                transpose(b)
                start_write(n, b)

        wait_write(n_slabs - 2, 0)
        wait_write(n_slabs - 1, 1)

    return gather_kernel


def kernel(x, table):
    NB, SEQ = x.shape
    V, D = table.shape
    idx = jnp.transpose(x).reshape(-1).astype(jnp.int32)
    out4 = _make_gather(NB, SEQ, V)(table, idx)
    out5 = out4.reshape(SEQ, D // 8, NB // _BT, 8, _BT)
    out = jnp.transpose(out5, (2, 4, 0, 1, 3)).reshape(NB, SEQ, D)
    return out


# speedup vs baseline: 1.8663x; 1.8663x over previous
import functools

import jax
import jax.numpy as jnp
from jax import lax
from jax.experimental import pallas as pl
from jax.experimental.pallas import tpu as pltpu
from jax.experimental.pallas import tpu_sc as plsc

_BT = 128
_D = 64


@functools.lru_cache(maxsize=None)
def _make_gather(NB, SEQ, V):
    info = plsc.get_sparse_core_info()
    NC, NS = info.num_cores, info.num_subcores
    NW = NC * NS
    assert NB % (NW * _BT) == 0
    bt_per_w = NB // _BT // NW
    n_slabs = bt_per_w * SEQ
    NBT = NB // _BT

    mesh = plsc.VectorSubcoreMesh(core_axis_name="c", subcore_axis_name="s")

    @functools.partial(
        pl.kernel,
        mesh=mesh,
        out_type=jax.ShapeDtypeStruct((SEQ, _D // 8, NBT, 8, _BT), jnp.float32),
        compiler_params=pltpu.CompilerParams(
            use_tc_tiling_on_sc=False, needs_layout_passes=False
        ),
        scratch_types=[
            pltpu.VMEM((2, _BT), jnp.int32),
            pltpu.VMEM((2, _BT, _D), jnp.float32),
            pltpu.VMEM((2, _D // 8, 8, _BT + 1), jnp.float32),
            pltpu.SemaphoreType.DMA,
            pltpu.SemaphoreType.DMA,
            pltpu.SemaphoreType.DMA,
            pltpu.SemaphoreType.DMA,
            pltpu.SemaphoreType.DMA,
            pltpu.SemaphoreType.DMA,
        ],
    )
    def gather_kernel(table_hbm, idx_hbm, out_hbm, sidx_v, rows_v, slab_v,
                      i0, i1, g0, g1, w0, w1):
        wid = lax.axis_index("s") * NC + lax.axis_index("c")
        bt0 = wid * bt_per_w
        isems = (i0, i1)
        gsems = (g0, g1)
        wsems = (w0, w1)
        iota = lax.iota(jnp.int32, 16)
        lhi = lax.shift_right_logical(iota, 3)
        ljr = lax.bitwise_and(iota, 7)
        zero16 = jnp.zeros((16,), jnp.int32)

        def slab_pos(n):
            bt_local = n // SEQ
            s = n - bt_local * SEQ
            return bt0 + bt_local, s

        def idx_slice(n):
            bt, s = slab_pos(n)
            return idx_hbm.at[pl.ds(s * NB + bt * _BT, _BT)]

        def start_idx(n, buf):
            pltpu.async_copy(idx_slice(n), sidx_v.at[buf], isems[buf])

        def wait_idx(n, buf):
            pltpu.make_async_copy(idx_slice(n), sidx_v.at[buf], isems[buf]).wait()

        def start_gather(buf):
            pltpu.async_copy(table_hbm.at[sidx_v.at[buf]], rows_v.at[buf], gsems[buf])

        def wait_gather(buf):
            pltpu.make_async_copy(
                table_hbm.at[sidx_v.at[buf]], rows_v.at[buf], gsems[buf]
            ).wait()

        def slab_src(buf):
            return slab_v.at[buf, :, :, pl.ds(0, _BT)]

        def out_slice(n):
            bt, s = slab_pos(n)
            return out_hbm.at[s, :, bt, :, :]

        def start_write(n, buf):
            pltpu.async_copy(slab_src(buf), out_slice(n), wsems[buf])

        def wait_write(n, buf):
            pltpu.make_async_copy(slab_src(buf), out_slice(n), wsems[buf]).wait()

        def transpose(buf):
            sbuf = slab_v.at[buf]

            @pl.loop(0, _BT, unroll=8)
            def _(bc):
                bcv = zero16 + bc
                for k in range(_D // 16):
                    vec = rows_v[buf, bc, pl.ds(k * 16, 16)]
                    plsc.store_scatter(sbuf, [lhi + 2 * k, ljr, bcv], vec)

        start_idx(0, 0)
        start_idx(1, 1)
        wait_idx(0, 0)
        start_gather(0)

        @pl.loop(0, n_slabs, step=2)
        def _(n0):
            for b in range(2):
                n = n0 + b
                nb = 1 - b
                wait_gather(b)

                @pl.when(n + 2 < n_slabs)
                def _():
                    start_idx(n + 2, b)

                @pl.when(n + 1 < n_slabs)
                def _():
                    wait_idx(n + 1, nb)
                    start_gather(nb)

                @pl.when(n >= 2)
                def _():
                    wait_write(n - 2, b)

                transpose(b)
                start_write(n, b)

        wait_write(n_slabs - 2, 0)
        wait_write(n_slabs - 1, 1)

    return gather_kernel


def kernel(x, table):
    NB, SEQ = x.shape
    V, D = table.shape
    idx = jnp.transpose(x).reshape(-1).astype(jnp.int32)
    out5 = _make_gather(NB, SEQ, V)(table, idx)
    out = jnp.transpose(out5, (2, 4, 0, 1, 3)).reshape(NB, SEQ, D)
    return out


# speedup vs baseline: 1.8676x; 1.0007x over previous
import functools

import jax
import jax.numpy as jnp
from jax import lax
from jax.experimental import pallas as pl
from jax.experimental.pallas import tpu as pltpu
from jax.experimental.pallas import tpu_sc as plsc

_BT = 128
_D = 64


@functools.lru_cache(maxsize=None)
def _make_gather(NB, SEQ, V):
    info = plsc.get_sparse_core_info()
    NC, NS = info.num_cores, info.num_subcores
    NW = NC * NS
    assert NB % (NW * _BT) == 0
    bt_per_w = NB // _BT // NW
    n_slabs = bt_per_w * SEQ
    NBT = NB // _BT

    mesh = plsc.VectorSubcoreMesh(core_axis_name="c", subcore_axis_name="s")

    @functools.partial(
        pl.kernel,
        mesh=mesh,
        out_type=jax.ShapeDtypeStruct((SEQ, _D // 8, NBT, 8, _BT), jnp.float32),
        compiler_params=pltpu.CompilerParams(
            use_tc_tiling_on_sc=False, needs_layout_passes=False
        ),
        scratch_types=[
            pltpu.VMEM((2, _BT), jnp.int32),
            pltpu.VMEM((2, _BT, _D), jnp.float32),
            pltpu.VMEM((2, _D // 8, 8, _BT + 1), jnp.float32),
            pltpu.SemaphoreType.DMA,
            pltpu.SemaphoreType.DMA,
            pltpu.SemaphoreType.DMA,
            pltpu.SemaphoreType.DMA,
            pltpu.SemaphoreType.DMA,
            pltpu.SemaphoreType.DMA,
        ],
    )
    def gather_kernel(table_hbm, idx_hbm, out_hbm, sidx_v, rows_v, slab_v,
                      i0, i1, g0, g1, w0, w1):
        wid = lax.axis_index("s") * NC + lax.axis_index("c")
        bt0 = wid * bt_per_w
        isems = (i0, i1)
        gsems = (g0, g1)
        wsems = (w0, w1)
        iota = lax.iota(jnp.int32, 16)
        lhi = lax.shift_right_logical(iota, 3)
        ljr = lax.bitwise_and(iota, 7)
        zero16 = jnp.zeros((16,), jnp.int32)

        def slab_pos(n):
            bt_local = n // SEQ
            s = n - bt_local * SEQ
            return bt0 + bt_local, s

        def idx_slice(n):
            bt, s = slab_pos(n)
            return idx_hbm.at[pl.ds(s * NB + bt * _BT, _BT)]

        def start_idx(n, buf):
            pltpu.async_copy(idx_slice(n), sidx_v.at[buf], isems[buf])

        def wait_idx(n, buf):
            pltpu.make_async_copy(idx_slice(n), sidx_v.at[buf], isems[buf]).wait()

        def start_gather(buf):
            pltpu.async_copy(table_hbm.at[sidx_v.at[buf]], rows_v.at[buf], gsems[buf])

        def wait_gather(buf):
            pltpu.make_async_copy(
                table_hbm.at[sidx_v.at[buf]], rows_v.at[buf], gsems[buf]
            ).wait()

        def slab_src(buf):
            return slab_v.at[buf, :, :, pl.ds(0, _BT)]

        def out_slice(n):
            bt, s = slab_pos(n)
            return out_hbm.at[s, :, bt, :, :]

        def start_write(n, buf):
            pltpu.async_copy(slab_src(buf), out_slice(n), wsems[buf])

        def wait_write(n, buf):
            pltpu.make_async_copy(slab_src(buf), out_slice(n), wsems[buf]).wait()

        def transpose(buf):
            sbuf = slab_v.at[buf]
            jtvs = [lhi + 2 * k for k in range(_D // 16)]

            @pl.loop(0, _BT, unroll=8)
            def _(bc):
                bcv = zero16 + bc
                for k in range(_D // 16):
                    vec = rows_v[buf, bc, pl.ds(k * 16, 16)]
                    plsc.store_scatter(sbuf, [jtvs[k], ljr, bcv], vec)

        start_idx(0, 0)
        start_idx(1, 1)
        wait_idx(0, 0)
        start_gather(0)

        @pl.loop(0, n_slabs, step=2)
        def _(n0):
            for b in range(2):
                n = n0 + b
                nb = 1 - b
                wait_gather(b)

                @pl.when(n + 2 < n_slabs)
                def _():
                    start_idx(n + 2, b)

                @pl.when(n + 1 < n_slabs)
                def _():
                    wait_idx(n + 1, nb)
                    start_gather(nb)

                @pl.when(n >= 2)
                def _():
                    wait_write(n - 2, b)

                transpose(b)
                start_write(n, b)

        wait_write(n_slabs - 2, 0)
        wait_write(n_slabs - 1, 1)

    return gather_kernel


def kernel(x, table):
    NB, SEQ = x.shape
    V, D = table.shape
    idx = jnp.transpose(x).reshape(-1).astype(jnp.int32)
    out5 = _make_gather(NB, SEQ, V)(table, idx)
    out = jnp.transpose(out5, (2, 4, 0, 1, 3)).reshape(NB, SEQ, D)
    return out
